# baseline (speedup 1.0000x reference)
import functools
import math

import jax
import jax.numpy as jnp
from jax import lax
from jax.experimental import pallas as pl
from jax.experimental.pallas import tpu as pltpu
from jax.experimental.pallas import tpu_sc as plsc

_B, _No, _Nm, _D, _H, _C, _K = 32, 1024, 512, 512, 1024, 400, 3
_EPS_LS = 0.001
_L = 16
_NG = _Nm // _L
_GPS = 4


def _sc_counts_body(sx_hbm, sy_hbm, mx_hbm, my_hbm, out_hbm,
                    sx_v, sy_v, mx_v, my_v, cnt_v):
    b = lax.axis_index("s") * 2 + lax.axis_index("c")
    pltpu.sync_copy(sx_hbm.at[b], sx_v)
    pltpu.sync_copy(sy_hbm.at[b], sy_v)
    pltpu.sync_copy(mx_hbm.at[b], mx_v)
    pltpu.sync_copy(my_hbm.at[b], my_v)

    zeros = jnp.zeros((_L,), jnp.float32)
    ones = jnp.ones((_L,), jnp.float32)
    lanes = lax.iota(jnp.int32, _L)

    def zero_body(o, zcarry):
        cnt_v[o] = zeros
        return zcarry
    lax.fori_loop(0, _No, zero_body, 0)

    big = jnp.full((_L,), 0xFFFFFFFF, jnp.uint32)
    hi_mask = jnp.uint32(0xFFFFFC00)
    lo_mask = jnp.int32(0x3FF)

    def sweep_body(gq, gcarry):
        g0 = gq * _GPS
        mxs = [mx_v[g0 + i] for i in range(_GPS)]
        mys = [my_v[g0 + i] for i in range(_GPS)]

        def p1(o16, carry):
            rs = list(carry)
            sxv = sx_v[o16]
            syv = sy_v[o16]
            for j in range(_L):
                sx = sxv[j]
                sy = syv[j]
                o = (o16 * _L + j).astype(jnp.uint32)
                for i in range(_GPS):
                    dx = mxs[i] - sx
                    dy = mys[i] - sy
                    d2 = dx * dx + dy * dy
                    key = (plsc.bitcast(d2, jnp.uint32) & hi_mask) | o
                    r1, r2, r3 = rs[i]
                    rs[i] = (jnp.minimum(r1, key),
                             jnp.minimum(r2, jnp.maximum(r1, key)),
                             jnp.minimum(r3, jnp.maximum(r2, key)))
            return tuple(rs)

        carry0 = tuple((big, big, big) for _ in range(_GPS))
        rs = lax.fori_loop(0, _No // _L, p1, carry0)
        for i in range(_GPS):
            for r in rs[i]:
                idx = plsc.bitcast(r, jnp.int32) & lo_mask
                plsc.addupdate_scatter(cnt_v, [idx, lanes], ones)
        return gcarry

    lax.fori_loop(0, _NG // _GPS, sweep_body, 0)
    pltpu.sync_copy(cnt_v, out_hbm.at[b])


def _counts16_sc(sx, sy, mx, my):
    mesh = plsc.VectorSubcoreMesh(core_axis_name="c", subcore_axis_name="s")
    return pl.kernel(
        _sc_counts_body,
        out_type=jax.ShapeDtypeStruct((_B, _No, _L), jnp.float32),
        mesh=mesh,
        compiler_params=pltpu.CompilerParams(use_tc_tiling_on_sc=False,
                                             needs_layout_passes=False),
        scratch_types=[
            pltpu.VMEM((_No // _L, _L), jnp.float32),
            pltpu.VMEM((_No // _L, _L), jnp.float32),
            pltpu.VMEM((_NG, _L), jnp.float32),
            pltpu.VMEM((_NG, _L), jnp.float32),
            pltpu.VMEM((_No, _L), jnp.float32),
        ],
    )(sx, sy, mx, my)


def _dense_body(mf_ref, sf_ref, c16_ref, wp_ref, bp_ref, wc_ref, bc_ref,
                lab_ref, logit_ref, loss_ref, x_scr):
    b = pl.program_id(0)
    m = mf_ref[0]
    s = sf_ref[0]
    c16 = c16_ref[0]
    msum = jnp.sum(m, axis=0) * (1.0 / _Nm)
    wT = lax.dot_general(c16, s, (((0,), (0,)), ((), ())),
                         preferred_element_type=jnp.float32)
    x = msum + jnp.sum(wT, axis=0) * (0.3 / (_Nm * _K))
    x_scr[pl.ds(b, 1), :] = x[None, :]

    @pl.when(b == _B - 1)
    def _head():
        xs = x_scr[...]
        h = jnp.dot(xs, wp_ref[...], preferred_element_type=jnp.float32)
        h = h + bp_ref[...]
        hg = 0.5 * h * (1.0 + lax.erf(h * (1.0 / math.sqrt(2.0))))
        logit = jnp.dot(hg, wc_ref[...], preferred_element_type=jnp.float32)
        logit = logit + bc_ref[...]
        mx = jnp.max(logit, axis=1, keepdims=True)
        ex = jnp.exp(logit - mx)
        lse = jnp.log(jnp.sum(ex, axis=1, keepdims=True)) + mx
        logp = logit - lse
        onehot = (lax.broadcasted_iota(jnp.int32, (_B, _C), 1) ==
                  lab_ref[...]).astype(jnp.float32)
        tgt = (1.0 - _EPS_LS) * onehot + _EPS_LS / _C
        loss = -jnp.sum(tgt * logp) * (1.0 / _B)
        logit_ref[...] = logit
        loss_ref[...] = jnp.reshape(loss, (1, 1))


def _dense_call(motion_feat, scene_feat, counts16,
                W_proj, b_proj, W_cls, b_cls, label):
    return pl.pallas_call(
        _dense_body,
        grid=(_B,),
        in_specs=[
            pl.BlockSpec((1, _Nm, _D), lambda b: (b, 0, 0)),
            pl.BlockSpec((1, _No, _D), lambda b: (b, 0, 0)),
            pl.BlockSpec((1, _No, _L), lambda b: (b, 0, 0)),
            pl.BlockSpec((_D, _H), lambda b: (0, 0)),
            pl.BlockSpec((1, _H), lambda b: (0, 0)),
            pl.BlockSpec((_H, _C), lambda b: (0, 0)),
            pl.BlockSpec((1, _C), lambda b: (0, 0)),
            pl.BlockSpec((_B, 1), lambda b: (0, 0)),
        ],
        out_specs=(
            pl.BlockSpec((_B, _C), lambda b: (0, 0)),
            pl.BlockSpec((1, 1), lambda b: (0, 0)),
        ),
        out_shape=(
            jax.ShapeDtypeStruct((_B, _C), jnp.float32),
            jax.ShapeDtypeStruct((1, 1), jnp.float32),
        ),
        scratch_shapes=[pltpu.VMEM((_B, _D), jnp.float32)],
    )(motion_feat, scene_feat, counts16, W_proj,
      b_proj.reshape(1, _H), W_cls, b_cls.reshape(1, _C),
      label.reshape(_B, 1))


def kernel(scene_feat, motion_feat, scene_loc, motion_loc, label,
           W_proj, b_proj, W_cls, b_cls):
    sx = scene_loc[:, :, 0].reshape(_B, _No // _L, _L)
    sy = scene_loc[:, :, 1].reshape(_B, _No // _L, _L)
    mx = motion_loc[:, :, 0].reshape(_B, _NG, _L)
    my = motion_loc[:, :, 1].reshape(_B, _NG, _L)
    counts16 = _counts16_sc(sx, sy, mx, my)
    logit, loss = _dense_call(motion_feat, scene_feat, counts16,
                              W_proj, b_proj, W_cls, b_cls, label)
    return (logit, loss.reshape(()))


# speedup vs baseline: 1.0373x; 1.0373x over previous
import functools
import math

import jax
import jax.numpy as jnp
from jax import lax
from jax.experimental import pallas as pl
from jax.experimental.pallas import tpu as pltpu
from jax.experimental.pallas import tpu_sc as plsc

_B, _No, _Nm, _D, _H, _C, _K = 32, 1024, 512, 512, 1024, 400, 3
_EPS_LS = 0.001
_L = 16
_NG = _Nm // _L
_GPS = 4
_BB = 2


def _sc_counts_body(sx_hbm, sy_hbm, mx_hbm, my_hbm, out_hbm,
                    sx_v, sy_v, mx_v, my_v, cnt_v):
    b = lax.axis_index("s") * 2 + lax.axis_index("c")
    pltpu.sync_copy(sx_hbm.at[b], sx_v)
    pltpu.sync_copy(sy_hbm.at[b], sy_v)
    pltpu.sync_copy(mx_hbm.at[b], mx_v)
    pltpu.sync_copy(my_hbm.at[b], my_v)

    zeros = jnp.zeros((_L,), jnp.float32)
    ones = jnp.ones((_L,), jnp.float32)
    lanes = lax.iota(jnp.int32, _L)

    def zero_body(o, zcarry):
        cnt_v[o] = zeros
        return zcarry
    lax.fori_loop(0, _No, zero_body, 0)

    big = jnp.full((_L,), 0xFFFFFFFF, jnp.uint32)
    hi_mask = jnp.uint32(0xFFFFFC00)
    lo_mask = jnp.int32(0x3FF)

    def sweep_body(gq, gcarry):
        g0 = gq * _GPS
        mxs = [mx_v[g0 + i] for i in range(_GPS)]
        mys = [my_v[g0 + i] for i in range(_GPS)]

        def p1(o16, carry):
            rs = list(carry)
            sxv = sx_v[o16]
            syv = sy_v[o16]
            for j in range(_L):
                sx = sxv[j]
                sy = syv[j]
                o = (o16 * _L + j).astype(jnp.uint32)
                for i in range(_GPS):
                    dx = mxs[i] - sx
                    dy = mys[i] - sy
                    d2 = dx * dx + dy * dy
                    key = (plsc.bitcast(d2, jnp.uint32) & hi_mask) | o
                    r1, r2, r3 = rs[i]
                    rs[i] = (jnp.minimum(r1, key),
                             jnp.minimum(r2, jnp.maximum(r1, key)),
                             jnp.minimum(r3, jnp.maximum(r2, key)))
            return tuple(rs)

        carry0 = tuple((big, big, big) for _ in range(_GPS))
        rs = lax.fori_loop(0, _No // _L, p1, carry0)
        for i in range(_GPS):
            for r in rs[i]:
                idx = plsc.bitcast(r, jnp.int32) & lo_mask
                plsc.addupdate_scatter(cnt_v, [idx, lanes], ones)
        return gcarry

    lax.fori_loop(0, _NG // _GPS, sweep_body, 0)
    pltpu.sync_copy(cnt_v, out_hbm.at[b])


def _counts16_sc(sx, sy, mx, my):
    mesh = plsc.VectorSubcoreMesh(core_axis_name="c", subcore_axis_name="s")
    return pl.kernel(
        _sc_counts_body,
        out_type=jax.ShapeDtypeStruct((_B, _No, _L), jnp.float32),
        mesh=mesh,
        compiler_params=pltpu.CompilerParams(use_tc_tiling_on_sc=False,
                                             needs_layout_passes=False),
        scratch_types=[
            pltpu.VMEM((_No // _L, _L), jnp.float32),
            pltpu.VMEM((_No // _L, _L), jnp.float32),
            pltpu.VMEM((_NG, _L), jnp.float32),
            pltpu.VMEM((_NG, _L), jnp.float32),
            pltpu.VMEM((_No, _L), jnp.float32),
        ],
    )(sx, sy, mx, my)


def _dense_body(mf_ref, sf_ref, c16_ref, wp_ref, bp_ref, wc_ref, bc_ref,
                lab_ref, logit_ref, loss_ref, x_scr):
    b = pl.program_id(0)
    for u in range(_BB):
        m = mf_ref[u]
        s = sf_ref[u]
        c16 = c16_ref[u]
        msum = jnp.sum(m, axis=0) * (1.0 / _Nm)
        wT = lax.dot_general(c16, s, (((0,), (0,)), ((), ())),
                             preferred_element_type=jnp.float32)
        x = msum + jnp.sum(wT, axis=0) * (0.3 / (_Nm * _K))
        x_scr[pl.ds(b * _BB + u, 1), :] = x[None, :]

    @pl.when(b == _B // _BB - 1)
    def _head():
        xs = x_scr[...]
        h = jnp.dot(xs, wp_ref[...], preferred_element_type=jnp.float32)
        h = h + bp_ref[...]
        hg = 0.5 * h * (1.0 + lax.erf(h * (1.0 / math.sqrt(2.0))))
        logit = jnp.dot(hg, wc_ref[...], preferred_element_type=jnp.float32)
        logit = logit + bc_ref[...]
        mx = jnp.max(logit, axis=1, keepdims=True)
        ex = jnp.exp(logit - mx)
        lse = jnp.log(jnp.sum(ex, axis=1, keepdims=True)) + mx
        logp = logit - lse
        onehot = (lax.broadcasted_iota(jnp.int32, (_B, _C), 1) ==
                  lab_ref[...]).astype(jnp.float32)
        tgt = (1.0 - _EPS_LS) * onehot + _EPS_LS / _C
        loss = -jnp.sum(tgt * logp) * (1.0 / _B)
        logit_ref[...] = logit
        loss_ref[...] = jnp.reshape(loss, (1, 1))


def _dense_call(motion_feat, scene_feat, counts16,
                W_proj, b_proj, W_cls, b_cls, label):
    return pl.pallas_call(
        _dense_body,
        grid=(_B // _BB,),
        in_specs=[
            pl.BlockSpec((_BB, _Nm, _D), lambda b: (b, 0, 0)),
            pl.BlockSpec((_BB, _No, _D), lambda b: (b, 0, 0)),
            pl.BlockSpec((_BB, _No, _L), lambda b: (b, 0, 0)),
            pl.BlockSpec((_D, _H), lambda b: (0, 0)),
            pl.BlockSpec((1, _H), lambda b: (0, 0)),
            pl.BlockSpec((_H, _C), lambda b: (0, 0)),
            pl.BlockSpec((1, _C), lambda b: (0, 0)),
            pl.BlockSpec((_B, 1), lambda b: (0, 0)),
        ],
        out_specs=(
            pl.BlockSpec((_B, _C), lambda b: (0, 0)),
            pl.BlockSpec((1, 1), lambda b: (0, 0)),
        ),
        out_shape=(
            jax.ShapeDtypeStruct((_B, _C), jnp.float32),
            jax.ShapeDtypeStruct((1, 1), jnp.float32),
        ),
        scratch_shapes=[pltpu.VMEM((_B, _D), jnp.float32)],
    )(motion_feat, scene_feat, counts16, W_proj,
      b_proj.reshape(1, _H), W_cls, b_cls.reshape(1, _C),
      label.reshape(_B, 1))


def kernel(scene_feat, motion_feat, scene_loc, motion_loc, label,
           W_proj, b_proj, W_cls, b_cls):
    sx = scene_loc[:, :, 0].reshape(_B, _No // _L, _L)
    sy = scene_loc[:, :, 1].reshape(_B, _No // _L, _L)
    mx = motion_loc[:, :, 0].reshape(_B, _NG, _L)
    my = motion_loc[:, :, 1].reshape(_B, _NG, _L)
    counts16 = _counts16_sc(sx, sy, mx, my)
    logit, loss = _dense_call(motion_feat, scene_feat, counts16,
                              W_proj, b_proj, W_cls, b_cls, label)
    return (logit, loss.reshape(()))


# speedup vs baseline: 1.0374x; 1.0001x over previous
import functools
import math

import jax
import jax.numpy as jnp
from jax import lax
from jax.experimental import pallas as pl
from jax.experimental.pallas import tpu as pltpu
from jax.experimental.pallas import tpu_sc as plsc

_B, _No, _Nm, _D, _H, _C, _K = 32, 1024, 512, 512, 1024, 400, 3
_EPS_LS = 0.001
_L = 16
_NG = _Nm // _L
_GPS = 4
_BB = 4


def _sc_counts_body(sx_hbm, sy_hbm, mx_hbm, my_hbm, out_hbm,
                    sx_v, sy_v, mx_v, my_v, cnt_v):
    b = lax.axis_index("s") * 2 + lax.axis_index("c")
    pltpu.sync_copy(sx_hbm.at[b], sx_v)
    pltpu.sync_copy(sy_hbm.at[b], sy_v)
    pltpu.sync_copy(mx_hbm.at[b], mx_v)
    pltpu.sync_copy(my_hbm.at[b], my_v)

    zeros = jnp.zeros((_L,), jnp.float32)
    ones = jnp.ones((_L,), jnp.float32)
    lanes = lax.iota(jnp.int32, _L)

    def zero_body(o, zcarry):
        cnt_v[o] = zeros
        return zcarry
    lax.fori_loop(0, _No, zero_body, 0)

    big = jnp.full((_L,), 0xFFFFFFFF, jnp.uint32)
    hi_mask = jnp.uint32(0xFFFFFC00)
    lo_mask = jnp.int32(0x3FF)

    def sweep_body(gq, gcarry):
        g0 = gq * _GPS
        mxs = [mx_v[g0 + i] for i in range(_GPS)]
        mys = [my_v[g0 + i] for i in range(_GPS)]

        def p1(o16, carry):
            rs = list(carry)
            sxv = sx_v[o16]
            syv = sy_v[o16]
            for j in range(_L):
                sx = sxv[j]
                sy = syv[j]
                o = (o16 * _L + j).astype(jnp.uint32)
                for i in range(_GPS):
                    dx = mxs[i] - sx
                    dy = mys[i] - sy
                    d2 = dx * dx + dy * dy
                    key = (plsc.bitcast(d2, jnp.uint32) & hi_mask) | o
                    r1, r2, r3 = rs[i]
                    rs[i] = (jnp.minimum(r1, key),
                             jnp.minimum(r2, jnp.maximum(r1, key)),
                             jnp.minimum(r3, jnp.maximum(r2, key)))
            return tuple(rs)

        carry0 = tuple((big, big, big) for _ in range(_GPS))
        rs = lax.fori_loop(0, _No // _L, p1, carry0)
        for i in range(_GPS):
            for r in rs[i]:
                idx = plsc.bitcast(r, jnp.int32) & lo_mask
                plsc.addupdate_scatter(cnt_v, [idx, lanes], ones)
        return gcarry

    lax.fori_loop(0, _NG // _GPS, sweep_body, 0)
    pltpu.sync_copy(cnt_v, out_hbm.at[b])


def _counts16_sc(sx, sy, mx, my):
    mesh = plsc.VectorSubcoreMesh(core_axis_name="c", subcore_axis_name="s")
    return pl.kernel(
        _sc_counts_body,
        out_type=jax.ShapeDtypeStruct((_B, _No, _L), jnp.float32),
        mesh=mesh,
        compiler_params=pltpu.CompilerParams(use_tc_tiling_on_sc=False,
                                             needs_layout_passes=False),
        scratch_types=[
            pltpu.VMEM((_No // _L, _L), jnp.float32),
            pltpu.VMEM((_No // _L, _L), jnp.float32),
            pltpu.VMEM((_NG, _L), jnp.float32),
            pltpu.VMEM((_NG, _L), jnp.float32),
            pltpu.VMEM((_No, _L), jnp.float32),
        ],
    )(sx, sy, mx, my)


def _dense_body(mf_ref, sf_ref, c16_ref, wp_ref, bp_ref, wc_ref, bc_ref,
                lab_ref, logit_ref, loss_ref, x_scr):
    b = pl.program_id(0)
    for u in range(_BB):
        m = mf_ref[u]
        s = sf_ref[u]
        c16 = c16_ref[u]
        msum = jnp.sum(m, axis=0) * (1.0 / _Nm)
        wT = lax.dot_general(c16, s, (((0,), (0,)), ((), ())),
                             preferred_element_type=jnp.float32)
        x = msum + jnp.sum(wT, axis=0) * (0.3 / (_Nm * _K))
        x_scr[pl.ds(b * _BB + u, 1), :] = x[None, :]

    @pl.when(b == _B // _BB - 1)
    def _head():
        xs = x_scr[...]
        h = jnp.dot(xs, wp_ref[...], preferred_element_type=jnp.float32)
        h = h + bp_ref[...]
        hg = 0.5 * h * (1.0 + lax.erf(h * (1.0 / math.sqrt(2.0))))
        logit = jnp.dot(hg, wc_ref[...], preferred_element_type=jnp.float32)
        logit = logit + bc_ref[...]
        mx = jnp.max(logit, axis=1, keepdims=True)
        ex = jnp.exp(logit - mx)
        lse = jnp.log(jnp.sum(ex, axis=1, keepdims=True)) + mx
        logp = logit - lse
        onehot = (lax.broadcasted_iota(jnp.int32, (_B, _C), 1) ==
                  lab_ref[...]).astype(jnp.float32)
        tgt = (1.0 - _EPS_LS) * onehot + _EPS_LS / _C
        loss = -jnp.sum(tgt * logp) * (1.0 / _B)
        logit_ref[...] = logit
        loss_ref[...] = jnp.reshape(loss, (1, 1))


def _dense_call(motion_feat, scene_feat, counts16,
                W_proj, b_proj, W_cls, b_cls, label):
    return pl.pallas_call(
        _dense_body,
        grid=(_B // _BB,),
        in_specs=[
            pl.BlockSpec((_BB, _Nm, _D), lambda b: (b, 0, 0)),
            pl.BlockSpec((_BB, _No, _D), lambda b: (b, 0, 0)),
            pl.BlockSpec((_BB, _No, _L), lambda b: (b, 0, 0)),
            pl.BlockSpec((_D, _H), lambda b: (0, 0)),
            pl.BlockSpec((1, _H), lambda b: (0, 0)),
            pl.BlockSpec((_H, _C), lambda b: (0, 0)),
            pl.BlockSpec((1, _C), lambda b: (0, 0)),
            pl.BlockSpec((_B, 1), lambda b: (0, 0)),
        ],
        out_specs=(
            pl.BlockSpec((_B, _C), lambda b: (0, 0)),
            pl.BlockSpec((1, 1), lambda b: (0, 0)),
        ),
        out_shape=(
            jax.ShapeDtypeStruct((_B, _C), jnp.float32),
            jax.ShapeDtypeStruct((1, 1), jnp.float32),
        ),
        scratch_shapes=[pltpu.VMEM((_B, _D), jnp.float32)],
    )(motion_feat, scene_feat, counts16, W_proj,
      b_proj.reshape(1, _H), W_cls, b_cls.reshape(1, _C),
      label.reshape(_B, 1))


def kernel(scene_feat, motion_feat, scene_loc, motion_loc, label,
           W_proj, b_proj, W_cls, b_cls):
    sx = scene_loc[:, :, 0].reshape(_B, _No // _L, _L)
    sy = scene_loc[:, :, 1].reshape(_B, _No // _L, _L)
    mx = motion_loc[:, :, 0].reshape(_B, _NG, _L)
    my = motion_loc[:, :, 1].reshape(_B, _NG, _L)
    counts16 = _counts16_sc(sx, sy, mx, my)
    logit, loss = _dense_call(motion_feat, scene_feat, counts16,
                              W_proj, b_proj, W_cls, b_cls, label)
    return (logit, loss.reshape(()))


# speedup vs baseline: 1.0894x; 1.0502x over previous
import functools
import math

import jax
import jax.numpy as jnp
from jax import lax
from jax.experimental import pallas as pl
from jax.experimental.pallas import tpu as pltpu
from jax.experimental.pallas import tpu_sc as plsc

_B, _No, _Nm, _D, _H, _C, _K = 32, 1024, 512, 512, 1024, 400, 3
_EPS_LS = 0.001
_L = 16
_NG = _Nm // _L
_GPS = 4
_BB = 4


def _sc_counts_body(sx_hbm, sy_hbm, mx_hbm, my_hbm, out_hbm,
                    sx_v, sy_v, mx_v, my_v, cnt_v):
    b = lax.axis_index("s") * 2 + lax.axis_index("c")
    pltpu.sync_copy(sx_hbm.at[b], sx_v)
    pltpu.sync_copy(sy_hbm.at[b], sy_v)
    pltpu.sync_copy(mx_hbm.at[b], mx_v)
    pltpu.sync_copy(my_hbm.at[b], my_v)

    zeros = jnp.zeros((_L,), jnp.float32)
    ones = jnp.ones((_L,), jnp.float32)
    lanes = lax.iota(jnp.int32, _L)

    def zero_body(o, zcarry):
        cnt_v[o] = zeros
        return zcarry
    lax.fori_loop(0, _No, zero_body, 0)

    big = jnp.full((_L,), 0xFFFFFFFF, jnp.uint32)
    hi_mask = jnp.uint32(0xFFFFFC00)
    lo_mask = jnp.int32(0x3FF)

    def sweep_body(gq, gcarry):
        g0 = gq * _GPS
        mxs = [mx_v[g0 + i] for i in range(_GPS)]
        mys = [my_v[g0 + i] for i in range(_GPS)]

        def p1(o16, carry):
            rs = list(carry)
            sxv = sx_v[o16]
            syv = sy_v[o16]
            for j in range(_L):
                sx = sxv[j]
                sy = syv[j]
                o = (o16 * _L + j).astype(jnp.uint32)
                for i in range(_GPS):
                    dx = mxs[i] - sx
                    dy = mys[i] - sy
                    d2 = dx * dx + dy * dy
                    key = (plsc.bitcast(d2, jnp.uint32) & hi_mask) | o
                    r1, r2, r3 = rs[i]
                    rs[i] = (jnp.minimum(r1, key),
                             jnp.minimum(r2, jnp.maximum(r1, key)),
                             jnp.minimum(r3, jnp.maximum(r2, key)))
            return tuple(rs)

        carry0 = tuple((big, big, big) for _ in range(_GPS))
        rs = lax.fori_loop(0, _No // _L, p1, carry0)
        for i in range(_GPS):
            for r in rs[i]:
                idx = plsc.bitcast(r, jnp.int32) & lo_mask
                plsc.addupdate_scatter(cnt_v, [idx, lanes], ones)
        return gcarry

    lax.fori_loop(0, _NG // _GPS, sweep_body, 0)
    pltpu.sync_copy(cnt_v, out_hbm.at[b])


def _counts16_sc(sx, sy, mx, my):
    mesh = plsc.VectorSubcoreMesh(core_axis_name="c", subcore_axis_name="s")
    return pl.kernel(
        _sc_counts_body,
        out_type=jax.ShapeDtypeStruct((_B, _No, _L), jnp.float32),
        mesh=mesh,
        compiler_params=pltpu.CompilerParams(use_tc_tiling_on_sc=False,
                                             needs_layout_passes=False),
        scratch_types=[
            pltpu.VMEM((_No // _L, _L), jnp.float32),
            pltpu.VMEM((_No // _L, _L), jnp.float32),
            pltpu.VMEM((_NG, _L), jnp.float32),
            pltpu.VMEM((_NG, _L), jnp.float32),
            pltpu.VMEM((_No, _L), jnp.float32),
        ],
    )(sx, sy, mx, my)


def _mean_body(mf_ref, x_ref):
    for u in range(_BB):
        x_ref[u] = (jnp.sum(mf_ref[u], axis=0) * (1.0 / _Nm))[None, :]


def _mean_call(motion_feat):
    return pl.pallas_call(
        _mean_body,
        grid=(_B // _BB,),
        in_specs=[pl.BlockSpec((_BB, _Nm, _D), lambda b: (b, 0, 0))],
        out_specs=pl.BlockSpec((_BB, 1, _D), lambda b: (b, 0, 0)),
        out_shape=jax.ShapeDtypeStruct((_B, 1, _D), jnp.float32),
    )(motion_feat)


def _dense_body(xm_ref, sf_ref, c16_ref, wp_ref, bp_ref, wc_ref, bc_ref,
                lab_ref, logit_ref, loss_ref, x_scr):
    b = pl.program_id(0)
    for u in range(_BB):
        s = sf_ref[u]
        c16 = c16_ref[u]
        msum = xm_ref[u, 0]
        wT = lax.dot_general(c16, s, (((0,), (0,)), ((), ())),
                             preferred_element_type=jnp.float32)
        x = msum + jnp.sum(wT, axis=0) * (0.3 / (_Nm * _K))
        x_scr[pl.ds(b * _BB + u, 1), :] = x[None, :]

    @pl.when(b == _B // _BB - 1)
    def _head():
        xs = x_scr[...]
        h = jnp.dot(xs, wp_ref[...], preferred_element_type=jnp.float32)
        h = h + bp_ref[...]
        hg = 0.5 * h * (1.0 + lax.erf(h * (1.0 / math.sqrt(2.0))))
        logit = jnp.dot(hg, wc_ref[...], preferred_element_type=jnp.float32)
        logit = logit + bc_ref[...]
        mx = jnp.max(logit, axis=1, keepdims=True)
        ex = jnp.exp(logit - mx)
        lse = jnp.log(jnp.sum(ex, axis=1, keepdims=True)) + mx
        logp = logit - lse
        onehot = (lax.broadcasted_iota(jnp.int32, (_B, _C), 1) ==
                  lab_ref[...]).astype(jnp.float32)
        tgt = (1.0 - _EPS_LS) * onehot + _EPS_LS / _C
        loss = -jnp.sum(tgt * logp) * (1.0 / _B)
        logit_ref[...] = logit
        loss_ref[...] = jnp.reshape(loss, (1, 1))


def _dense_call(xm, scene_feat, counts16,
                W_proj, b_proj, W_cls, b_cls, label):
    return pl.pallas_call(
        _dense_body,
        grid=(_B // _BB,),
        in_specs=[
            pl.BlockSpec((_BB, 1, _D), lambda b: (b, 0, 0)),
            pl.BlockSpec((_BB, _No, _D), lambda b: (b, 0, 0)),
            pl.BlockSpec((_BB, _No, _L), lambda b: (b, 0, 0)),
            pl.BlockSpec((_D, _H), lambda b: (0, 0)),
            pl.BlockSpec((1, _H), lambda b: (0, 0)),
            pl.BlockSpec((_H, _C), lambda b: (0, 0)),
            pl.BlockSpec((1, _C), lambda b: (0, 0)),
            pl.BlockSpec((_B, 1), lambda b: (0, 0)),
        ],
        out_specs=(
            pl.BlockSpec((_B, _C), lambda b: (0, 0)),
            pl.BlockSpec((1, 1), lambda b: (0, 0)),
        ),
        out_shape=(
            jax.ShapeDtypeStruct((_B, _C), jnp.float32),
            jax.ShapeDtypeStruct((1, 1), jnp.float32),
        ),
        scratch_shapes=[pltpu.VMEM((_B, _D), jnp.float32)],
    )(xm, scene_feat, counts16, W_proj,
      b_proj.reshape(1, _H), W_cls, b_cls.reshape(1, _C),
      label.reshape(_B, 1))


def kernel(scene_feat, motion_feat, scene_loc, motion_loc, label,
           W_proj, b_proj, W_cls, b_cls):
    sx = scene_loc[:, :, 0].reshape(_B, _No // _L, _L)
    sy = scene_loc[:, :, 1].reshape(_B, _No // _L, _L)
    mx = motion_loc[:, :, 0].reshape(_B, _NG, _L)
    my = motion_loc[:, :, 1].reshape(_B, _NG, _L)
    xm = _mean_call(motion_feat)
    counts16 = _counts16_sc(sx, sy, mx, my)
    logit, loss = _dense_call(xm, scene_feat, counts16,
                              W_proj, b_proj, W_cls, b_cls, label)
    return (logit, loss.reshape(()))


# speedup vs baseline: 1.1871x; 1.0896x over previous
import functools
import math

import jax
import jax.numpy as jnp
from jax import lax
from jax.experimental import pallas as pl
from jax.experimental.pallas import tpu as pltpu
from jax.experimental.pallas import tpu_sc as plsc

_B, _No, _Nm, _D, _H, _C, _K = 32, 1024, 512, 512, 1024, 400, 3
_EPS_LS = 0.001
_L = 16
_NG = _Nm // _L
_GPS = 4
_BB = 4


def _sc_counts_body(sx_hbm, sy_hbm, mx_hbm, my_hbm, out_hbm,
                    sx_v, sy_v, mx_v, my_v, cnt_v, keys_v):
    b = lax.axis_index("s") * 2 + lax.axis_index("c")
    pltpu.sync_copy(sx_hbm.at[b], sx_v)
    pltpu.sync_copy(sy_hbm.at[b], sy_v)
    pltpu.sync_copy(mx_hbm.at[b], mx_v)
    pltpu.sync_copy(my_hbm.at[b], my_v)

    zeros = jnp.zeros((_L,), jnp.float32)
    ones = jnp.ones((_L,), jnp.float32)
    lanes = lax.iota(jnp.int32, _L)

    def zero_body(o, zcarry):
        cnt_v[o] = zeros
        return zcarry
    lax.fori_loop(0, _No, zero_body, 0)

    big = jnp.full((_L,), 0xFFFFFFFF, jnp.uint32)
    hi_mask = jnp.uint32(0xFFFFFC00)
    lo_mask = jnp.int32(0x3FF)

    chunk_mask = jnp.int32(0x3F0)

    def sweep_body(gq, gcarry):
        g0 = gq * _GPS
        mxs = [mx_v[g0 + i] for i in range(_GPS)]
        mys = [my_v[g0 + i] for i in range(_GPS)]

        def p1(o16, carry):
            cs = list(carry)
            sxv = sx_v[o16]
            syv = sy_v[o16]
            cmins = [None] * _GPS
            for j in range(_L):
                sx = sxv[j]
                sy = syv[j]
                o = (o16 * _L + j).astype(jnp.uint32)
                for i in range(_GPS):
                    dx = mxs[i] - sx
                    dy = mys[i] - sy
                    d2 = dx * dx + dy * dy
                    key = (plsc.bitcast(d2, jnp.uint32) & hi_mask) | o
                    keys_v[i * _No + o16 * _L + j] = plsc.bitcast(
                        key, jnp.float32)
                    cmins[i] = key if j == 0 else jnp.minimum(cmins[i], key)
            for i in range(_GPS):
                c1, c2, c3 = cs[i]
                k = cmins[i]
                cs[i] = (jnp.minimum(c1, k),
                         jnp.minimum(c2, jnp.maximum(c1, k)),
                         jnp.minimum(c3, jnp.maximum(c2, k)))
            return tuple(cs)

        carry0 = tuple((big, big, big) for _ in range(_GPS))
        cs = lax.fori_loop(0, _No // _L, p1, carry0)

        for i in range(_GPS):
            bases = [(plsc.bitcast(c, jnp.int32) & chunk_mask) + i * _No
                     for c in cs[i]]
            r1 = r2 = r3 = big
            for base in bases:
                for j in range(_L):
                    kf = plsc.load_gather(keys_v, [base + j, lanes])
                    key = plsc.bitcast(kf, jnp.uint32)
                    r1, r2, r3 = (jnp.minimum(r1, key),
                                  jnp.minimum(r2, jnp.maximum(r1, key)),
                                  jnp.minimum(r3, jnp.maximum(r2, key)))
            for r in (r1, r2, r3):
                idx = plsc.bitcast(r, jnp.int32) & lo_mask
                plsc.addupdate_scatter(cnt_v, [idx, lanes], ones)
        return gcarry

    lax.fori_loop(0, _NG // _GPS, sweep_body, 0)
    pltpu.sync_copy(cnt_v, out_hbm.at[b])


def _counts16_sc(sx, sy, mx, my):
    mesh = plsc.VectorSubcoreMesh(core_axis_name="c", subcore_axis_name="s")
    return pl.kernel(
        _sc_counts_body,
        out_type=jax.ShapeDtypeStruct((_B, _No, _L), jnp.float32),
        mesh=mesh,
        compiler_params=pltpu.CompilerParams(use_tc_tiling_on_sc=False,
                                             needs_layout_passes=False),
        scratch_types=[
            pltpu.VMEM((_No // _L, _L), jnp.float32),
            pltpu.VMEM((_No // _L, _L), jnp.float32),
            pltpu.VMEM((_NG, _L), jnp.float32),
            pltpu.VMEM((_NG, _L), jnp.float32),
            pltpu.VMEM((_No, _L), jnp.float32),
            pltpu.VMEM((_GPS * _No, _L), jnp.float32),
        ],
    )(sx, sy, mx, my)


def _mean_body(mf_ref, x_ref):
    for u in range(_BB):
        x_ref[u] = (jnp.sum(mf_ref[u], axis=0) * (1.0 / _Nm))[None, :]


def _mean_call(motion_feat):
    return pl.pallas_call(
        _mean_body,
        grid=(_B // _BB,),
        in_specs=[pl.BlockSpec((_BB, _Nm, _D), lambda b: (b, 0, 0))],
        out_specs=pl.BlockSpec((_BB, 1, _D), lambda b: (b, 0, 0)),
        out_shape=jax.ShapeDtypeStruct((_B, 1, _D), jnp.float32),
    )(motion_feat)


def _dense_body(xm_ref, sf_ref, c16_ref, wp_ref, bp_ref, wc_ref, bc_ref,
                lab_ref, logit_ref, loss_ref, x_scr):
    b = pl.program_id(0)
    for u in range(_BB):
        s = sf_ref[u]
        c16 = c16_ref[u]
        msum = xm_ref[u, 0]
        wT = lax.dot_general(c16, s, (((0,), (0,)), ((), ())),
                             preferred_element_type=jnp.float32)
        x = msum + jnp.sum(wT, axis=0) * (0.3 / (_Nm * _K))
        x_scr[pl.ds(b * _BB + u, 1), :] = x[None, :]

    @pl.when(b == _B // _BB - 1)
    def _head():
        xs = x_scr[...]
        h = jnp.dot(xs, wp_ref[...], preferred_element_type=jnp.float32)
        h = h + bp_ref[...]
        hg = 0.5 * h * (1.0 + lax.erf(h * (1.0 / math.sqrt(2.0))))
        logit = jnp.dot(hg, wc_ref[...], preferred_element_type=jnp.float32)
        logit = logit + bc_ref[...]
        mx = jnp.max(logit, axis=1, keepdims=True)
        ex = jnp.exp(logit - mx)
        lse = jnp.log(jnp.sum(ex, axis=1, keepdims=True)) + mx
        logp = logit - lse
        onehot = (lax.broadcasted_iota(jnp.int32, (_B, _C), 1) ==
                  lab_ref[...]).astype(jnp.float32)
        tgt = (1.0 - _EPS_LS) * onehot + _EPS_LS / _C
        loss = -jnp.sum(tgt * logp) * (1.0 / _B)
        logit_ref[...] = logit
        loss_ref[...] = jnp.reshape(loss, (1, 1))


def _dense_call(xm, scene_feat, counts16,
                W_proj, b_proj, W_cls, b_cls, label):
    return pl.pallas_call(
        _dense_body,
        grid=(_B // _BB,),
        in_specs=[
            pl.BlockSpec((_BB, 1, _D), lambda b: (b, 0, 0)),
            pl.BlockSpec((_BB, _No, _D), lambda b: (b, 0, 0)),
            pl.BlockSpec((_BB, _No, _L), lambda b: (b, 0, 0)),
            pl.BlockSpec((_D, _H), lambda b: (0, 0)),
            pl.BlockSpec((1, _H), lambda b: (0, 0)),
            pl.BlockSpec((_H, _C), lambda b: (0, 0)),
            pl.BlockSpec((1, _C), lambda b: (0, 0)),
            pl.BlockSpec((_B, 1), lambda b: (0, 0)),
        ],
        out_specs=(
            pl.BlockSpec((_B, _C), lambda b: (0, 0)),
            pl.BlockSpec((1, 1), lambda b: (0, 0)),
        ),
        out_shape=(
            jax.ShapeDtypeStruct((_B, _C), jnp.float32),
            jax.ShapeDtypeStruct((1, 1), jnp.float32),
        ),
        scratch_shapes=[pltpu.VMEM((_B, _D), jnp.float32)],
    )(xm, scene_feat, counts16, W_proj,
      b_proj.reshape(1, _H), W_cls, b_cls.reshape(1, _C),
      label.reshape(_B, 1))


def kernel(scene_feat, motion_feat, scene_loc, motion_loc, label,
           W_proj, b_proj, W_cls, b_cls):
    sx = scene_loc[:, :, 0].reshape(_B, _No // _L, _L)
    sy = scene_loc[:, :, 1].reshape(_B, _No // _L, _L)
    mx = motion_loc[:, :, 0].reshape(_B, _NG, _L)
    my = motion_loc[:, :, 1].reshape(_B, _NG, _L)
    xm = _mean_call(motion_feat)
    counts16 = _counts16_sc(sx, sy, mx, my)
    logit, loss = _dense_call(xm, scene_feat, counts16,
                              W_proj, b_proj, W_cls, b_cls, label)
    return (logit, loss.reshape(()))


# speedup vs baseline: 1.2450x; 1.0488x over previous
import functools
import math

import jax
import jax.numpy as jnp
from jax import lax
from jax.experimental import pallas as pl
from jax.experimental.pallas import tpu as pltpu
from jax.experimental.pallas import tpu_sc as plsc

_B, _No, _Nm, _D, _H, _C, _K = 32, 1024, 512, 512, 1024, 400, 3
_EPS_LS = 0.001
_L = 16
_NG = _Nm // _L
_GPS = 4
_BB = 4


def _sc_counts_body(sx_hbm, sy_hbm, mx_hbm, my_hbm, out_hbm,
                    sx_v, sy_v, mx_v, my_v, cnt_v, keys_v):
    b = lax.axis_index("s") * 2 + lax.axis_index("c")
    pltpu.sync_copy(sx_hbm.at[b], sx_v)
    pltpu.sync_copy(sy_hbm.at[b], sy_v)
    pltpu.sync_copy(mx_hbm.at[b], mx_v)
    pltpu.sync_copy(my_hbm.at[b], my_v)

    zeros = jnp.zeros((_L,), jnp.float32)
    ones = jnp.ones((_L,), jnp.float32)
    lanes = lax.iota(jnp.int32, _L)

    def zero_body(o, zcarry):
        cnt_v[o] = zeros
        return zcarry
    lax.fori_loop(0, _No, zero_body, 0)

    big = jnp.full((_L,), 0xFFFFFFFF, jnp.uint32)
    hi_mask = jnp.uint32(0xFFFFFC00)
    lo_mask = jnp.int32(0x3FF)

    cid_hi = jnp.uint32(0xFFFFFFC0)
    cid_lo = jnp.int32(0x3F)

    def sweep_body(gq, gcarry):
        g0 = gq * _GPS
        mxs = [mx_v[g0 + i] for i in range(_GPS)]
        mys = [my_v[g0 + i] for i in range(_GPS)]

        def p1(o16, carry):
            cs = list(carry)
            sxv = sx_v[o16]
            syv = sy_v[o16]
            cmins = [None] * _GPS
            for j in range(_L):
                sx = sxv[j]
                sy = syv[j]
                for i in range(_GPS):
                    dx = mxs[i] - sx
                    dy = mys[i] - sy
                    d2 = dx * dx + dy * dy
                    keys_v[i * _No + o16 * _L + j] = d2
                    cmins[i] = d2 if j == 0 else jnp.minimum(cmins[i], d2)
            cid = o16.astype(jnp.uint32)
            for i in range(_GPS):
                c1, c2, c3 = cs[i]
                k = (plsc.bitcast(cmins[i], jnp.uint32) & cid_hi) | cid
                cs[i] = (jnp.minimum(c1, k),
                         jnp.minimum(c2, jnp.maximum(c1, k)),
                         jnp.minimum(c3, jnp.maximum(c2, k)))
            return tuple(cs)

        carry0 = tuple((big, big, big) for _ in range(_GPS))
        cs = lax.fori_loop(0, _No // _L, p1, carry0)

        for i in range(_GPS):
            r1 = r2 = r3 = big
            for c in cs[i]:
                sbase = (plsc.bitcast(c, jnp.int32) & cid_lo) * _L
                rbase = sbase + i * _No
                for j in range(_L):
                    kf = plsc.load_gather(keys_v, [rbase + j, lanes])
                    key = ((plsc.bitcast(kf, jnp.uint32) & hi_mask) |
                           (sbase + j).astype(jnp.uint32))
                    r1, r2, r3 = (jnp.minimum(r1, key),
                                  jnp.minimum(r2, jnp.maximum(r1, key)),
                                  jnp.minimum(r3, jnp.maximum(r2, key)))
            for r in (r1, r2, r3):
                idx = plsc.bitcast(r, jnp.int32) & lo_mask
                plsc.addupdate_scatter(cnt_v, [idx, lanes], ones)
        return gcarry

    lax.fori_loop(0, _NG // _GPS, sweep_body, 0)
    pltpu.sync_copy(cnt_v, out_hbm.at[b])


def _counts16_sc(sx, sy, mx, my):
    mesh = plsc.VectorSubcoreMesh(core_axis_name="c", subcore_axis_name="s")
    return pl.kernel(
        _sc_counts_body,
        out_type=jax.ShapeDtypeStruct((_B, _No, _L), jnp.float32),
        mesh=mesh,
        compiler_params=pltpu.CompilerParams(use_tc_tiling_on_sc=False,
                                             needs_layout_passes=False),
        scratch_types=[
            pltpu.VMEM((_No // _L, _L), jnp.float32),
            pltpu.VMEM((_No // _L, _L), jnp.float32),
            pltpu.VMEM((_NG, _L), jnp.float32),
            pltpu.VMEM((_NG, _L), jnp.float32),
            pltpu.VMEM((_No, _L), jnp.float32),
            pltpu.VMEM((_GPS * _No, _L), jnp.float32),
        ],
    )(sx, sy, mx, my)


def _mean_body(mf_ref, x_ref):
    for u in range(_BB):
        x_ref[u] = (jnp.sum(mf_ref[u], axis=0) * (1.0 / _Nm))[None, :]


def _mean_call(motion_feat):
    return pl.pallas_call(
        _mean_body,
        grid=(_B // _BB,),
        in_specs=[pl.BlockSpec((_BB, _Nm, _D), lambda b: (b, 0, 0))],
        out_specs=pl.BlockSpec((_BB, 1, _D), lambda b: (b, 0, 0)),
        out_shape=jax.ShapeDtypeStruct((_B, 1, _D), jnp.float32),
    )(motion_feat)


def _dense_body(xm_ref, sf_ref, c16_ref, wp_ref, bp_ref, wc_ref, bc_ref,
                lab_ref, logit_ref, loss_ref, x_scr):
    b = pl.program_id(0)
    for u in range(_BB):
        s = sf_ref[u]
        c16 = c16_ref[u]
        msum = xm_ref[u, 0]
        wT = lax.dot_general(c16, s, (((0,), (0,)), ((), ())),
                             preferred_element_type=jnp.float32)
        x = msum + jnp.sum(wT, axis=0) * (0.3 / (_Nm * _K))
        x_scr[pl.ds(b * _BB + u, 1), :] = x[None, :]

    @pl.when(b == _B // _BB - 1)
    def _head():
        xs = x_scr[...]
        h = jnp.dot(xs, wp_ref[...], preferred_element_type=jnp.float32)
        h = h + bp_ref[...]
        hg = 0.5 * h * (1.0 + lax.erf(h * (1.0 / math.sqrt(2.0))))
        logit = jnp.dot(hg, wc_ref[...], preferred_element_type=jnp.float32)
        logit = logit + bc_ref[...]
        mx = jnp.max(logit, axis=1, keepdims=True)
        ex = jnp.exp(logit - mx)
        lse = jnp.log(jnp.sum(ex, axis=1, keepdims=True)) + mx
        logp = logit - lse
        onehot = (lax.broadcasted_iota(jnp.int32, (_B, _C), 1) ==
                  lab_ref[...]).astype(jnp.float32)
        tgt = (1.0 - _EPS_LS) * onehot + _EPS_LS / _C
        loss = -jnp.sum(tgt * logp) * (1.0 / _B)
        logit_ref[...] = logit
        loss_ref[...] = jnp.reshape(loss, (1, 1))


def _dense_call(xm, scene_feat, counts16,
                W_proj, b_proj, W_cls, b_cls, label):
    return pl.pallas_call(
        _dense_body,
        grid=(_B // _BB,),
        in_specs=[
            pl.BlockSpec((_BB, 1, _D), lambda b: (b, 0, 0)),
            pl.BlockSpec((_BB, _No, _D), lambda b: (b, 0, 0)),
            pl.BlockSpec((_BB, _No, _L), lambda b: (b, 0, 0)),
            pl.BlockSpec((_D, _H), lambda b: (0, 0)),
            pl.BlockSpec((1, _H), lambda b: (0, 0)),
            pl.BlockSpec((_H, _C), lambda b: (0, 0)),
            pl.BlockSpec((1, _C), lambda b: (0, 0)),
            pl.BlockSpec((_B, 1), lambda b: (0, 0)),
        ],
        out_specs=(
            pl.BlockSpec((_B, _C), lambda b: (0, 0)),
            pl.BlockSpec((1, 1), lambda b: (0, 0)),
        ),
        out_shape=(
            jax.ShapeDtypeStruct((_B, _C), jnp.float32),
            jax.ShapeDtypeStruct((1, 1), jnp.float32),
        ),
        scratch_shapes=[pltpu.VMEM((_B, _D), jnp.float32)],
    )(xm, scene_feat, counts16, W_proj,
      b_proj.reshape(1, _H), W_cls, b_cls.reshape(1, _C),
      label.reshape(_B, 1))


def kernel(scene_feat, motion_feat, scene_loc, motion_loc, label,
           W_proj, b_proj, W_cls, b_cls):
    sx = scene_loc[:, :, 0].reshape(_B, _No // _L, _L)
    sy = scene_loc[:, :, 1].reshape(_B, _No // _L, _L)
    mx = motion_loc[:, :, 0].reshape(_B, _NG, _L)
    my = motion_loc[:, :, 1].reshape(_B, _NG, _L)
    xm = _mean_call(motion_feat)
    counts16 = _counts16_sc(sx, sy, mx, my)
    logit, loss = _dense_call(xm, scene_feat, counts16,
                              W_proj, b_proj, W_cls, b_cls, label)
    return (logit, loss.reshape(()))
